# Initial kernel scaffold; baseline (speedup 1.0000x reference)
import jax
import jax.numpy as jnp
from jax.experimental import pallas as pl


def kernel(x_nchw, w1, b1, gamma, beta, w2, b2):
    raise NotImplementedError("write your pallas kernel here")


# speedup vs baseline: 1.2328x; 1.2328x over previous
import jax
import jax.numpy as jnp
from jax import lax
from jax.experimental import pallas as pl
from jax.experimental.pallas import tpu as pltpu

_BN_EPS = 1e-5
_C = 3

_PAIRS = ((0, 0), (0, 1), (0, 2), (1, 1), (1, 2), (2, 2))
_NSTAT = _C + len(_PAIRS)
_LANE = 128
_SUB = 8
_NPARAM = 2 * _C + 3
_TARGET_BLOCK_BYTES = 3 * 1024 * 1024


def _round_up(v, m):
    return -(-v // m) * m


def _part_sum(a):
    lane = a.shape[-1]
    rows = a.size // lane
    return a.reshape(rows // _SUB, _SUB, lane).sum(axis=0)


def _plan_tiles(rows, n):
    per_sample = _C * rows * _LANE * 4
    if per_sample <= _TARGET_BLOCK_BYTES:
        s = rows
        nb = 1
        want = max(1, _TARGET_BLOCK_BYTES // per_sample)
        for d in range(1, n + 1):
            if n % d == 0 and d <= want:
                nb = d
    else:
        nb = 1
        s = _SUB
        cap = _TARGET_BLOCK_BYTES // (_C * _LANE * 4)
        for cand in range(_SUB, rows + 1, _SUB):
            if rows % cand == 0 and cand <= cap:
                s = cand
    return nb, s


def _fused_forward(x_nchw, w1, b1, gamma, beta, w2, b2):
    del b1
    N, c_in, H, W = x_nchw.shape
    assert c_in == _C
    HW = H * W
    M = N * HW
    inv_m = 1.0 / float(M)

    HWp = _round_up(HW, _LANE * _SUB)
    ROWS = HWp // _LANE

    x3 = x_nchw.reshape(N, _C, HW).astype(jnp.float32)
    if HWp != HW:
        x3 = jnp.pad(x3, ((0, 0), (0, 0), (0, HWp - HW)))
    x4 = x3.reshape(N, _C, ROWS, _LANE)

    Nb, S = _plan_tiles(ROWS, N)
    tn = N // Nb
    tr = ROWS // S
    T = tn * tr

    w1f = w1.astype(jnp.float32)
    w2f = w2.astype(jnp.float32)
    params = jnp.concatenate(
        [w1f, w2f,
         gamma.astype(jnp.float32)[:, None],
         beta.astype(jnp.float32)[:, None],
         b2.astype(jnp.float32)[:, None]], axis=1)

    def body(p_ref, x_ref, o_ref, acc_ref, wb_ref):
        ph = pl.program_id(0)
        t = pl.program_id(1) * tr + pl.program_id(2)

        @pl.when(jnp.logical_and(ph == 0, t == 0))
        def _init():
            acc_ref[...] = jnp.zeros_like(acc_ref)

        @pl.when(ph == 0)
        def _stats():
            xs = [x_ref[:, c:c + 1, :, :] for c in range(_C)]
            parts = [_part_sum(xs[c]) for c in range(_C)]
            parts += [_part_sum(xs[i] * xs[j]) for (i, j) in _PAIRS]
            acc_ref[...] += jnp.stack(parts, axis=0)

        @pl.when(jnp.logical_and(ph == 0, t == T - 1))
        def _fold():
            tot = [jnp.sum(acc_ref[k]) for k in range(_NSTAT)]
            mean = [tot[c] * inv_m for c in range(_C)]
            exx = {}
            for k, (i, j) in enumerate(_PAIRS):
                exx[(i, j)] = tot[_C + k] * inv_m
                exx[(j, i)] = exx[(i, j)]
            cov = [[exx[(i, j)] - mean[i] * mean[j] for j in range(_C)]
                   for i in range(_C)]
            w1s = [[p_ref[i, j] for j in range(_C)] for i in range(_C)]
            w2s = [[p_ref[i, _C + j] for j in range(_C)] for i in range(_C)]
            g = []
            for c in range(_C):
                vh = sum(w1s[c][i] * cov[i][j] * w1s[c][j]
                         for i in range(_C) for j in range(_C))
                vh = jnp.maximum(vh, 0.0) + _BN_EPS
                rs = lax.rsqrt(jnp.full((1, _LANE), vh, jnp.float32))[0, 0]
                g.append(p_ref[c, 2 * _C] * rs)
            for c in range(_C):
                for j in range(_C):
                    wb_ref[c, j] = sum(w2s[c][k] * g[k] * w1s[k][j]
                                       for k in range(_C))
                mh = [sum(w1s[k][i] * mean[i] for i in range(_C))
                      for k in range(_C)]
                wb_ref[c, _C] = p_ref[c, 2 * _C + 2] + sum(
                    w2s[c][k] * (p_ref[k, 2 * _C + 1] - g[k] * mh[k])
                    for k in range(_C))

        @pl.when(ph == 1)
        def _apply():
            xs = [x_ref[:, c:c + 1, :, :] for c in range(_C)]
            for c in range(_C):
                o_ref[:, c:c + 1, :, :] = (
                    wb_ref[c, 0] * xs[0] + wb_ref[c, 1] * xs[1]
                    + wb_ref[c, 2] * xs[2] + wb_ref[c, _C])

    x_spec = pl.BlockSpec((Nb, _C, S, _LANE), lambda p, n, r: (n, 0, r, 0))
    o_spec = pl.BlockSpec(
        (Nb, _C, S, _LANE),
        lambda p, n, r: (jnp.where(p == 0, 0, n), 0,
                         jnp.where(p == 0, 0, r), 0))
    p_spec = pl.BlockSpec((_C, _NPARAM), lambda p, n, r: (0, 0),
                          memory_space=pltpu.MemorySpace.SMEM)

    out4 = pl.pallas_call(
        body,
        out_shape=jax.ShapeDtypeStruct((N, _C, ROWS, _LANE), jnp.float32),
        grid=(2, tn, tr),
        in_specs=[p_spec, x_spec],
        out_specs=o_spec,
        scratch_shapes=[pltpu.VMEM((_NSTAT, _SUB, _LANE), jnp.float32),
                        pltpu.SMEM((_C, _C + 1), jnp.float32)],
        compiler_params=pltpu.CompilerParams(
            dimension_semantics=("arbitrary", "arbitrary", "arbitrary"),
            vmem_limit_bytes=64 * 1024 * 1024),
        cost_estimate=pl.CostEstimate(
            flops=33 * M, transcendentals=0, bytes_accessed=12 * _C * M),
    )(params, x4)

    out3 = out4.reshape(N, _C, HWp)
    if HWp != HW:
        out3 = out3[:, :, :HW]
    return out3.reshape(N, _C, H, W)


def kernel(x_nchw, w1, b1, gamma, beta, w2, b2):
    return _fused_forward(x_nchw, w1, b1, gamma, beta, w2, b2)


# speedup vs baseline: 1.3626x; 1.1053x over previous
import jax
import jax.numpy as jnp
from jax import lax
from jax.experimental import pallas as pl
from jax.experimental.pallas import tpu as pltpu

_BN_EPS = 1e-5
_C = 3

_PAIRS = ((0, 0), (0, 1), (0, 2), (1, 1), (1, 2), (2, 2))
_NSTAT = _C + len(_PAIRS)
_LANE = 128
_SUB = 8
_NPARAM = 2 * _C + 3
_TARGET_BLOCK_BYTES = 3 * 1024 * 1024


def _round_up(v, m):
    return -(-v // m) * m


def _part_sum(a):
    lane = a.shape[-1]
    rows = a.size // lane
    return a.reshape(rows // _SUB, _SUB, lane).sum(axis=0)


def _plan_tiles(rows, n):
    per_sample = _C * rows * _LANE * 4
    if per_sample <= _TARGET_BLOCK_BYTES:
        s = rows
        nb = 1
        want = max(1, _TARGET_BLOCK_BYTES // per_sample)
        for d in range(1, n + 1):
            if n % d == 0 and d <= want:
                nb = d
    else:
        nb = 1
        s = _SUB
        cap = _TARGET_BLOCK_BYTES // (_C * _LANE * 4)
        for cand in range(_SUB, rows + 1, _SUB):
            if rows % cand == 0 and cand <= cap:
                s = cand
    return nb, s


def _fused_forward(x_nchw, w1, b1, gamma, beta, w2, b2):
    del b1
    N, c_in, H, W = x_nchw.shape
    assert c_in == _C
    HW = H * W
    M = N * HW
    inv_m = 1.0 / float(M)

    HWp = _round_up(HW, _LANE * _SUB)
    ROWS = HWp // _LANE

    x3 = x_nchw.reshape(N, _C, HW).astype(jnp.float32)
    if HWp != HW:
        x3 = jnp.pad(x3, ((0, 0), (0, 0), (0, HWp - HW)))
    x4 = x3.reshape(N, _C, ROWS, _LANE)

    Nb, S = _plan_tiles(ROWS, N)
    tn = N // Nb
    tr = ROWS // S
    T = tn * tr
    resident = N * _C * ROWS * _LANE * 4 <= 40 * 1024 * 1024

    w1f = w1.astype(jnp.float32)
    w2f = w2.astype(jnp.float32)
    params = jnp.concatenate(
        [w1f, w2f,
         gamma.astype(jnp.float32)[:, None],
         beta.astype(jnp.float32)[:, None],
         b2.astype(jnp.float32)[:, None]], axis=1)

    def body(p_ref, x_ref, o_ref, acc_ref, wb_ref, xbuf_ref):
        ph = pl.program_id(0)
        n = pl.program_id(1)
        r = pl.program_id(2)
        t = n * tr + r

        @pl.when(jnp.logical_and(ph == 0, t == 0))
        def _init():
            acc_ref[...] = jnp.zeros_like(acc_ref)

        @pl.when(ph == 0)
        def _stats():
            xv = x_ref[...]
            if resident:
                xbuf_ref[pl.ds(n * Nb, Nb), :, pl.ds(r * S, S), :] = xv
            xs = [xv[:, c:c + 1, :, :] for c in range(_C)]
            parts = [_part_sum(xs[c]) for c in range(_C)]
            parts += [_part_sum(xs[i] * xs[j]) for (i, j) in _PAIRS]
            acc_ref[...] += jnp.stack(parts, axis=0)

        @pl.when(jnp.logical_and(ph == 0, t == T - 1))
        def _fold():
            tot = [jnp.sum(acc_ref[k]) for k in range(_NSTAT)]
            mean = [tot[c] * inv_m for c in range(_C)]
            exx = {}
            for k, (i, j) in enumerate(_PAIRS):
                exx[(i, j)] = tot[_C + k] * inv_m
                exx[(j, i)] = exx[(i, j)]
            cov = [[exx[(i, j)] - mean[i] * mean[j] for j in range(_C)]
                   for i in range(_C)]
            w1s = [[p_ref[i, j] for j in range(_C)] for i in range(_C)]
            w2s = [[p_ref[i, _C + j] for j in range(_C)] for i in range(_C)]
            g = []
            for c in range(_C):
                vh = sum(w1s[c][i] * cov[i][j] * w1s[c][j]
                         for i in range(_C) for j in range(_C))
                vh = jnp.maximum(vh, 0.0) + _BN_EPS
                rs = lax.rsqrt(jnp.full((1, _LANE), vh, jnp.float32))[0, 0]
                g.append(p_ref[c, 2 * _C] * rs)
            for c in range(_C):
                for j in range(_C):
                    wb_ref[c, j] = sum(w2s[c][k] * g[k] * w1s[k][j]
                                       for k in range(_C))
                mh = [sum(w1s[k][i] * mean[i] for i in range(_C))
                      for k in range(_C)]
                wb_ref[c, _C] = p_ref[c, 2 * _C + 2] + sum(
                    w2s[c][k] * (p_ref[k, 2 * _C + 1] - g[k] * mh[k])
                    for k in range(_C))

        @pl.when(ph == 1)
        def _apply():
            if resident:
                xv = xbuf_ref[pl.ds(n * Nb, Nb), :, pl.ds(r * S, S), :]
            else:
                xv = x_ref[...]
            xs = [xv[:, c:c + 1, :, :] for c in range(_C)]
            for c in range(_C):
                o_ref[:, c:c + 1, :, :] = (
                    wb_ref[c, 0] * xs[0] + wb_ref[c, 1] * xs[1]
                    + wb_ref[c, 2] * xs[2] + wb_ref[c, _C])

    if resident:
        x_spec = pl.BlockSpec(
            (Nb, _C, S, _LANE),
            lambda p, n, r: (jnp.where(p == 0, n, tn - 1), 0,
                             jnp.where(p == 0, r, tr - 1), 0))
    else:
        x_spec = pl.BlockSpec((Nb, _C, S, _LANE),
                              lambda p, n, r: (n, 0, r, 0))
    o_spec = pl.BlockSpec(
        (Nb, _C, S, _LANE),
        lambda p, n, r: (jnp.where(p == 0, 0, n), 0,
                         jnp.where(p == 0, 0, r), 0))
    p_spec = pl.BlockSpec((_C, _NPARAM), lambda p, n, r: (0, 0),
                          memory_space=pltpu.MemorySpace.SMEM)

    out4 = pl.pallas_call(
        body,
        out_shape=jax.ShapeDtypeStruct((N, _C, ROWS, _LANE), jnp.float32),
        grid=(2, tn, tr),
        in_specs=[p_spec, x_spec],
        out_specs=o_spec,
        scratch_shapes=[pltpu.VMEM((_NSTAT, _SUB, _LANE), jnp.float32),
                        pltpu.SMEM((_C, _C + 1), jnp.float32),
                        pltpu.VMEM((N, _C, ROWS, _LANE) if resident
                                   else (1, 1, _SUB, _LANE), jnp.float32)],
        compiler_params=pltpu.CompilerParams(
            dimension_semantics=("arbitrary", "arbitrary", "arbitrary"),
            vmem_limit_bytes=64 * 1024 * 1024),
        cost_estimate=pl.CostEstimate(
            flops=33 * M, transcendentals=0, bytes_accessed=12 * _C * M),
    )(params, x4)

    out3 = out4.reshape(N, _C, HWp)
    if HWp != HW:
        out3 = out3[:, :, :HW]
    return out3.reshape(N, _C, H, W)


def kernel(x_nchw, w1, b1, gamma, beta, w2, b2):
    return _fused_forward(x_nchw, w1, b1, gamma, beta, w2, b2)


# speedup vs baseline: 1.4039x; 1.0303x over previous
import jax
import jax.numpy as jnp
from jax import lax
from jax.experimental import pallas as pl
from jax.experimental.pallas import tpu as pltpu

_BN_EPS = 1e-5
_C = 3

_PAIRS = ((0, 0), (0, 1), (0, 2), (1, 1), (1, 2), (2, 2))
_NSTAT = _C + len(_PAIRS)
_LANE = 128
_SUB = 8
_NPARAM = 2 * _C + 3
_TARGET_BLOCK_BYTES = 6 * 1024 * 1024


def _round_up(v, m):
    return -(-v // m) * m


def _part_sum(a):
    lane = a.shape[-1]
    rows = a.size // lane
    return a.reshape(rows // _SUB, _SUB, lane).sum(axis=0)


def _plan_tiles(rows, n):
    per_sample = _C * rows * _LANE * 4
    if per_sample <= _TARGET_BLOCK_BYTES:
        s = rows
        nb = 1
        want = max(1, _TARGET_BLOCK_BYTES // per_sample)
        for d in range(1, n + 1):
            if n % d == 0 and d <= want:
                nb = d
    else:
        nb = 1
        s = _SUB
        cap = _TARGET_BLOCK_BYTES // (_C * _LANE * 4)
        for cand in range(_SUB, rows + 1, _SUB):
            if rows % cand == 0 and cand <= cap:
                s = cand
    return nb, s


def _fused_forward(x_nchw, w1, b1, gamma, beta, w2, b2):
    del b1
    N, c_in, H, W = x_nchw.shape
    assert c_in == _C
    HW = H * W
    M = N * HW
    inv_m = 1.0 / float(M)

    HWp = _round_up(HW, _LANE * _SUB)
    ROWS = HWp // _LANE

    x3 = x_nchw.reshape(N, _C, HW).astype(jnp.float32)
    if HWp != HW:
        x3 = jnp.pad(x3, ((0, 0), (0, 0), (0, HWp - HW)))
    x4 = x3.reshape(N, _C, ROWS, _LANE)

    Nb, S = _plan_tiles(ROWS, N)
    tn = N // Nb
    tr = ROWS // S
    T = tn * tr
    resident = N * _C * ROWS * _LANE * 4 <= 40 * 1024 * 1024

    w1f = w1.astype(jnp.float32)
    w2f = w2.astype(jnp.float32)
    params = jnp.concatenate(
        [w1f, w2f,
         gamma.astype(jnp.float32)[:, None],
         beta.astype(jnp.float32)[:, None],
         b2.astype(jnp.float32)[:, None]], axis=1)

    def body(p_ref, x_ref, o_ref, acc_ref, wb_ref, xbuf_ref):
        ph = pl.program_id(0)
        n = pl.program_id(1)
        r = pl.program_id(2)
        t = n * tr + r

        @pl.when(jnp.logical_and(ph == 0, t == 0))
        def _init():
            acc_ref[...] = jnp.zeros_like(acc_ref)

        @pl.when(ph == 0)
        def _stats():
            xv = x_ref[...]
            if resident:
                xbuf_ref[pl.ds(n * Nb, Nb), :, pl.ds(r * S, S), :] = xv
            xs = [xv[:, c:c + 1, :, :] for c in range(_C)]
            parts = [_part_sum(xs[c]) for c in range(_C)]
            parts += [_part_sum(xs[i] * xs[j]) for (i, j) in _PAIRS]
            acc_ref[...] += jnp.stack(parts, axis=0)

        @pl.when(jnp.logical_and(ph == 0, t == T - 1))
        def _fold():
            tot = [jnp.sum(acc_ref[k]) for k in range(_NSTAT)]
            mean = [tot[c] * inv_m for c in range(_C)]
            exx = {}
            for k, (i, j) in enumerate(_PAIRS):
                exx[(i, j)] = tot[_C + k] * inv_m
                exx[(j, i)] = exx[(i, j)]
            cov = [[exx[(i, j)] - mean[i] * mean[j] for j in range(_C)]
                   for i in range(_C)]
            w1s = [[p_ref[i, j] for j in range(_C)] for i in range(_C)]
            w2s = [[p_ref[i, _C + j] for j in range(_C)] for i in range(_C)]
            g = []
            for c in range(_C):
                vh = sum(w1s[c][i] * cov[i][j] * w1s[c][j]
                         for i in range(_C) for j in range(_C))
                vh = jnp.maximum(vh, 0.0) + _BN_EPS
                rs = lax.rsqrt(jnp.full((1, _LANE), vh, jnp.float32))[0, 0]
                g.append(p_ref[c, 2 * _C] * rs)
            for c in range(_C):
                for j in range(_C):
                    wb_ref[c, j] = sum(w2s[c][k] * g[k] * w1s[k][j]
                                       for k in range(_C))
                mh = [sum(w1s[k][i] * mean[i] for i in range(_C))
                      for k in range(_C)]
                wb_ref[c, _C] = p_ref[c, 2 * _C + 2] + sum(
                    w2s[c][k] * (p_ref[k, 2 * _C + 1] - g[k] * mh[k])
                    for k in range(_C))

        @pl.when(ph == 1)
        def _apply():
            if resident:
                xv = xbuf_ref[pl.ds(n * Nb, Nb), :, pl.ds(r * S, S), :]
            else:
                xv = x_ref[...]
            xs = [xv[:, c:c + 1, :, :] for c in range(_C)]
            for c in range(_C):
                o_ref[:, c:c + 1, :, :] = (
                    wb_ref[c, 0] * xs[0] + wb_ref[c, 1] * xs[1]
                    + wb_ref[c, 2] * xs[2] + wb_ref[c, _C])

    if resident:
        x_spec = pl.BlockSpec(
            (Nb, _C, S, _LANE),
            lambda p, n, r: (jnp.where(p == 0, n, tn - 1), 0,
                             jnp.where(p == 0, r, tr - 1), 0))
    else:
        x_spec = pl.BlockSpec((Nb, _C, S, _LANE),
                              lambda p, n, r: (n, 0, r, 0))
    o_spec = pl.BlockSpec(
        (Nb, _C, S, _LANE),
        lambda p, n, r: (jnp.where(p == 0, 0, n), 0,
                         jnp.where(p == 0, 0, r), 0))
    p_spec = pl.BlockSpec((_C, _NPARAM), lambda p, n, r: (0, 0),
                          memory_space=pltpu.MemorySpace.SMEM)

    out4 = pl.pallas_call(
        body,
        out_shape=jax.ShapeDtypeStruct((N, _C, ROWS, _LANE), jnp.float32),
        grid=(2, tn, tr),
        in_specs=[p_spec, x_spec],
        out_specs=o_spec,
        scratch_shapes=[pltpu.VMEM((_NSTAT, _SUB, _LANE), jnp.float32),
                        pltpu.SMEM((_C, _C + 1), jnp.float32),
                        pltpu.VMEM((N, _C, ROWS, _LANE) if resident
                                   else (1, 1, _SUB, _LANE), jnp.float32)],
        compiler_params=pltpu.CompilerParams(
            dimension_semantics=("arbitrary", "arbitrary", "arbitrary"),
            vmem_limit_bytes=64 * 1024 * 1024),
        cost_estimate=pl.CostEstimate(
            flops=33 * M, transcendentals=0, bytes_accessed=12 * _C * M),
    )(params, x4)

    out3 = out4.reshape(N, _C, HWp)
    if HWp != HW:
        out3 = out3[:, :, :HW]
    return out3.reshape(N, _C, H, W)


def kernel(x_nchw, w1, b1, gamma, beta, w2, b2):
    return _fused_forward(x_nchw, w1, b1, gamma, beta, w2, b2)
